# baseline (speedup 1.0000x reference)
import functools
import math

import jax
import jax.numpy as jnp
from jax import lax
from jax.experimental import pallas as pl
from jax.experimental.pallas import tpu as pltpu
from jax.experimental.pallas import tpu_sc as plsc

_NW = 32
_CHUNK = 128
_LANES = 16
_NBUF = 6
_LOOK = 3


def _emb_body(idx_hbm, table_hbm, out_hbm, idx_v, *rest, n_chunks, d):
    bufs = rest[:_NBUF]
    gsem = rest[_NBUF:2 * _NBUF]
    wsem = rest[2 * _NBUF:3 * _NBUF]
    n_per_w = n_chunks * _CHUNK
    wid = lax.axis_index("s") * 2 + lax.axis_index("c")
    base = wid * n_per_w
    pltpu.sync_copy(idx_hbm.at[pl.ds(base, n_per_w)], idx_v)
    scale = jnp.full((_LANES,), math.sqrt(d), dtype=jnp.float32)
    vecs_per_row = d // _LANES

    def start_gather(j, b):
        pltpu.async_copy(
            table_hbm.at[idx_v.at[pl.ds(j * _CHUNK, _CHUNK)]], bufs[b], gsem[b])

    def wait_gather(b):
        pltpu.make_async_copy(
            table_hbm.at[idx_v.at[pl.ds(0, _CHUNK)]], bufs[b], gsem[b]).wait()

    def wait_write(b):
        pltpu.make_async_copy(bufs[b], out_hbm.at[pl.ds(base, _CHUNK)],
                              wsem[b]).wait()

    def scale_and_write(j, b):
        buf = bufs[b]
        half = _CHUNK // 2

        def row_body(i, _):
            r = i * 2
            for rr in (r, r + 1):
                for v in range(vecs_per_row):
                    sl = pl.ds(v * _LANES, _LANES)
                    buf[rr, sl] = buf[rr, sl] * scale
            return 0

        for h in range(2):
            lax.fori_loop(h * half // 2, (h + 1) * half // 2, row_body, 0,
                          unroll=2)
            pltpu.async_copy(
                buf.at[pl.ds(h * half, half)],
                out_hbm.at[pl.ds(base + j * _CHUNK + h * half, half)],
                wsem[b])

    def process(j, b, wait_w, issue_g):
        wait_gather(b)
        if issue_g:
            bg = (b + _LOOK) % _NBUF
            if wait_w:
                wait_write(bg)
            start_gather(j + _LOOK, bg)
        scale_and_write(j, b)

    for j in range(_LOOK):
        start_gather(j, j % _NBUF)
    for j in range(_NBUF - _LOOK):
        process(j, j % _NBUF, wait_w=False, issue_g=True)

    j_start = _NBUF - _LOOK
    n_steady = n_chunks - _NBUF
    n_outer = n_steady // _NBUF

    def outer(k, _):
        j0 = j_start + _NBUF * k
        for i in range(_NBUF):
            process(j0 + i, (j_start + i) % _NBUF, wait_w=True, issue_g=True)
        return 0

    lax.fori_loop(0, n_outer, outer, 0)
    j_tail = j_start + _NBUF * n_outer
    for t in range(n_steady % _NBUF):
        process(j_tail + t, (j_tail + t) % _NBUF, wait_w=True, issue_g=True)

    for j in range(n_chunks - _LOOK, n_chunks):
        process(j, j % _NBUF, wait_w=False, issue_g=False)
    for b in range(_NBUF):
        wait_write(b)


def kernel(input, table):
    b, l = input.shape
    v, d = table.shape
    n = b * l
    assert n % (_NW * _CHUNK) == 0
    n_chunks = n // (_NW * _CHUNK)
    assert n_chunks >= 2 * _NBUF

    idx = input.T.reshape(n).astype(jnp.int32)
    mesh = plsc.VectorSubcoreMesh(core_axis_name="c", subcore_axis_name="s")

    emb = pl.kernel(
        functools.partial(_emb_body, n_chunks=n_chunks, d=d),
        mesh=mesh,
        compiler_params=pltpu.CompilerParams(use_tc_tiling_on_sc=True),
        out_type=jax.ShapeDtypeStruct((n, d), jnp.float32),
        scratch_types=(
            [pltpu.VMEM((n_chunks * _CHUNK,), jnp.int32)]
            + [pltpu.VMEM((_CHUNK, d), jnp.float32)] * _NBUF
            + [pltpu.SemaphoreType.DMA] * (2 * _NBUF)
        ),
    )(idx, table)
    return emb.reshape(l, b, d).transpose(1, 0, 2)


# speedup vs baseline: 1.0260x; 1.0260x over previous
import functools
import math

import jax
import jax.numpy as jnp
from jax import lax
from jax.experimental import pallas as pl
from jax.experimental.pallas import tpu as pltpu
from jax.experimental.pallas import tpu_sc as plsc

_NW = 32
_CHUNK = 128
_LANES = 16
_NBUF = 6
_LOOK = 3


def _emb_body(idx_hbm, table_hbm, out_hbm, idx_v, *rest, n_chunks, d):
    bufs = rest[:_NBUF]
    gsem = rest[_NBUF:2 * _NBUF]
    wsem = rest[2 * _NBUF:3 * _NBUF]
    n_per_w = n_chunks * _CHUNK
    wid = lax.axis_index("s") * 2 + lax.axis_index("c")
    base = wid * n_per_w
    pltpu.sync_copy(idx_hbm.at[pl.ds(base, n_per_w)], idx_v)
    scale = jnp.full((_LANES,), math.sqrt(d), dtype=jnp.float32)
    vecs_per_row = d // _LANES

    def start_gather(j, b):
        pltpu.async_copy(
            table_hbm.at[idx_v.at[pl.ds(j * _CHUNK, _CHUNK)]], bufs[b], gsem[b])

    def wait_gather(b):
        pltpu.make_async_copy(
            table_hbm.at[idx_v.at[pl.ds(0, _CHUNK)]], bufs[b], gsem[b]).wait()

    def wait_write(b):
        pltpu.make_async_copy(bufs[b], out_hbm.at[pl.ds(base, _CHUNK)],
                              wsem[b]).wait()

    def scale_and_write(j, b):
        buf = bufs[b]

        def row_body(i, _):
            r = i * 2
            for rr in (r, r + 1):
                for v in range(vecs_per_row):
                    sl = pl.ds(v * _LANES, _LANES)
                    buf[rr, sl] = buf[rr, sl] * scale
            return 0

        lax.fori_loop(0, _CHUNK // 2, row_body, 0, unroll=2)
        pltpu.async_copy(buf, out_hbm.at[pl.ds(base + j * _CHUNK, _CHUNK)],
                         wsem[b])

    def process(j, b, wait_w, issue_g):
        wait_gather(b)
        if issue_g:
            bg = (b + _LOOK) % _NBUF
            if wait_w:
                wait_write(bg)
            start_gather(j + _LOOK, bg)
        scale_and_write(j, b)

    for j in range(_LOOK):
        start_gather(j, j % _NBUF)
    for j in range(_NBUF - _LOOK):
        process(j, j % _NBUF, wait_w=False, issue_g=True)

    j_start = _NBUF - _LOOK
    n_steady = n_chunks - _NBUF
    n_outer = n_steady // _NBUF

    def outer(k, _):
        j0 = j_start + _NBUF * k
        for i in range(_NBUF):
            process(j0 + i, (j_start + i) % _NBUF, wait_w=True, issue_g=True)
        return 0

    lax.fori_loop(0, n_outer, outer, 0)
    j_tail = j_start + _NBUF * n_outer
    for t in range(n_steady % _NBUF):
        process(j_tail + t, (j_tail + t) % _NBUF, wait_w=True, issue_g=True)

    for j in range(n_chunks - _LOOK, n_chunks):
        process(j, j % _NBUF, wait_w=False, issue_g=False)
    for b in range(_NBUF):
        wait_write(b)


def kernel(input, table):
    b, l = input.shape
    v, d = table.shape
    n = b * l
    assert n % (_NW * _CHUNK) == 0
    n_chunks = n // (_NW * _CHUNK)
    assert n_chunks >= 2 * _NBUF

    idx = input.T.reshape(n).astype(jnp.int32)
    mesh = plsc.VectorSubcoreMesh(core_axis_name="c", subcore_axis_name="s")

    emb = pl.kernel(
        functools.partial(_emb_body, n_chunks=n_chunks, d=d),
        mesh=mesh,
        compiler_params=pltpu.CompilerParams(use_tc_tiling_on_sc=True),
        out_type=jax.ShapeDtypeStruct((n, d), jnp.float32),
        scratch_types=(
            [pltpu.VMEM((n_chunks * _CHUNK,), jnp.int32)]
            + [pltpu.VMEM((_CHUNK, d), jnp.float32)] * _NBUF
            + [pltpu.SemaphoreType.DMA] * (2 * _NBUF)
        ),
    )(idx, table)
    return emb.reshape(l, b, d).transpose(1, 0, 2)
